# Initial kernel scaffold; baseline (speedup 1.0000x reference)
import jax
import jax.numpy as jnp
from jax.experimental import pallas as pl


def kernel(hidden_states, pooled_output, proj_W1, proj_b1, proj_W2, proj_b2, gat1_W, gat1_a, gat2_W, gat2_a, lin_W, lin_b):
    raise NotImplementedError("write your pallas kernel here")


# speedup vs baseline: 22.2642x; 22.2642x over previous
import jax
import jax.numpy as jnp
from jax.experimental import pallas as pl
from jax.experimental.pallas import tpu as pltpu

BS = 4
SEQ = 4096
TILE = 1024
NT = SEQ // TILE
STEPS = 2 * NT + 1
EPS = 1e-16
NEG = -1e30


def _tm(x):
    return jnp.sqrt(1.0 + jnp.sum(x * x, axis=-1, keepdims=True))


def _lrelu(x):
    return jnp.where(x >= 0, x, 0.2 * x)


def _body(hs_ref, ps_ref, W1_ref, b1_ref, W2t_ref, W2s_ref, b2_ref,
          g1Wt_ref, g1Ws_ref, a1d_ref, a1s_ref,
          g2Wt_ref, g2Ws_ref, a2d_ref, a2s_ref,
          linT_ref, linS_ref, linb_ref,
          out_ref, gm_ref,
          h2_buf, u2_buf, v2_buf, stats, scal):
    s = pl.program_id(1)

    ps = ps_ref[0]
    pt = _tm(ps)

    g1Wt = g1Wt_ref[...]
    g2Wt = g2Wt_ref[...]
    a1d = a1d_ref[...]
    a1s = a1s_ref[...]
    a2d = a2d_ref[...]
    a2s = a2s_ref[...]

    h1_0 = pt * g1Wt + jnp.dot(ps, g1Ws_ref[...])
    u1_0 = jnp.sum(h1_0 * a1d, axis=1, keepdims=True)
    v1_0 = jnp.sum(h1_0 * a1s, axis=1, keepdims=True)

    @pl.when(s == 0)
    def _init1():
        scal[0:1, :] = jnp.full((1, 1), NEG, jnp.float32)
        scal[1:2, :] = jnp.zeros((1, 1), jnp.float32)
        stats[0:1, :] = jnp.zeros((1, 384), jnp.float32)

    @pl.when(s < NT)
    def _pass1():
        x = hs_ref[0, 0]
        y1 = jnp.dot(x, W1_ref[...], preferred_element_type=jnp.float32)
        y1 = y1 + b1_ref[...]
        g = jax.nn.gelu(y1)
        tg = _tm(g)
        y2 = tg * W2t_ref[...] + jnp.dot(g, W2s_ref[...],
                                         preferred_element_type=jnp.float32)
        y2 = y2 + b2_ref[...]
        t2 = _tm(y2)
        h1 = t2 * g1Wt + jnp.dot(y2, g1Ws_ref[...],
                                 preferred_element_type=jnp.float32)
        u1 = jnp.sum(h1 * a1d, axis=1, keepdims=True)
        v1 = jnp.sum(h1 * a1s, axis=1, keepdims=True)

        e0 = _lrelu(u1 + v1_0)
        es = _lrelu(u1 + v1)
        mm = jnp.maximum(e0, es)
        w0 = jnp.exp(e0 - mm)
        ws = jnp.exp(es - mm)
        agg1 = (w0 * h1_0 + ws * h1) / (w0 + ws + EPS)

        z = jax.nn.gelu(agg1)
        tz = _tm(z)
        h2 = tz * g2Wt + jnp.dot(z, g2Ws_ref[...],
                                 preferred_element_type=jnp.float32)
        u2 = jnp.sum(h2 * a2d, axis=1, keepdims=True)
        v2 = jnp.sum(h2 * a2s, axis=1, keepdims=True)

        off = s * TILE
        h2_buf[pl.ds(off, TILE), :] = h2
        u2_buf[pl.ds(off, TILE), :] = u2
        v2_buf[pl.ds(off, TILE), :] = v2

        sc = _lrelu(u1_0 + v1)
        mt = jnp.max(sc, keepdims=True)
        m_old = scal[0:1, :]
        m_new = jnp.maximum(m_old, mt)
        c = jnp.exp(m_old - m_new)
        p = jnp.exp(sc - m_new)
        scal[0:1, :] = m_new
        scal[1:2, :] = scal[1:2, :] * c + jnp.sum(p, keepdims=True)
        stats[0:1, :] = stats[0:1, :] * c + jax.lax.dot_general(
            p, h1, (((0,), (0,)), ((), ())),
            preferred_element_type=jnp.float32)

    @pl.when(s == NT)
    def _hub():
        e_self = _lrelu(u1_0 + v1_0)
        m1 = scal[0:1, :]
        m_f = jnp.maximum(m1, e_self)
        l = scal[1:2, :] * jnp.exp(m1 - m_f) + jnp.exp(e_self - m_f)
        acc = stats[0:1, :] * jnp.exp(m1 - m_f) + jnp.exp(e_self - m_f) * h1_0
        agg1_0 = acc / (l + EPS)

        z0 = jax.nn.gelu(agg1_0)
        tz0 = _tm(z0)
        h2_0 = tz0 * g2Wt + jnp.dot(z0, g2Ws_ref[...],
                                    preferred_element_type=jnp.float32)
        stats[3:4, :] = h2_0
        scal[4:5, :] = jnp.sum(h2_0 * a2d, axis=1, keepdims=True)
        scal[5:6, :] = jnp.sum(h2_0 * a2s, axis=1, keepdims=True)

        scal[2:3, :] = jnp.full((1, 1), NEG, jnp.float32)
        scal[3:4, :] = jnp.zeros((1, 1), jnp.float32)
        scal[6:7, :] = jnp.zeros((1, 1), jnp.float32)
        stats[1:2, :] = jnp.zeros((1, 384), jnp.float32)
        stats[2:3, :] = jnp.zeros((1, 384), jnp.float32)

    @pl.when(s > NT)
    def _pass2():
        off = (s - NT - 1) * TILE
        h2 = h2_buf[pl.ds(off, TILE), :]
        u2 = u2_buf[pl.ds(off, TILE), :]
        v2 = v2_buf[pl.ds(off, TILE), :]
        h2_0 = stats[3:4, :]
        u2_0 = scal[4:5, :]
        v2_0 = scal[5:6, :]

        sc = _lrelu(u2_0 + v2)
        mt = jnp.max(sc, keepdims=True)
        m_old = scal[2:3, :]
        m_new = jnp.maximum(m_old, mt)
        c = jnp.exp(m_old - m_new)
        p = jnp.exp(sc - m_new)
        scal[2:3, :] = m_new
        scal[3:4, :] = scal[3:4, :] * c + jnp.sum(p, keepdims=True)
        stats[1:2, :] = stats[1:2, :] * c + jax.lax.dot_general(
            p, h2, (((0,), (0,)), ((), ())),
            preferred_element_type=jnp.float32)

        e0 = _lrelu(u2 + v2_0)
        es = _lrelu(u2 + v2)
        mm = jnp.maximum(e0, es)
        w0 = jnp.exp(e0 - mm)
        ws = jnp.exp(es - mm)
        agg2 = (w0 * h2_0 + ws * h2) / (w0 + ws + EPS)
        tt = _tm(agg2)
        stats[2:3, :] = stats[2:3, :] + jnp.sum(agg2, axis=0, keepdims=True)
        scal[6:7, :] = scal[6:7, :] + jnp.sum(tt, keepdims=True)

    @pl.when(s == STEPS - 1)
    def _final():
        h2_0 = stats[3:4, :]
        u2_0 = scal[4:5, :]
        v2_0 = scal[5:6, :]
        e_self = _lrelu(u2_0 + v2_0)
        m2 = scal[2:3, :]
        m_f = jnp.maximum(m2, e_self)
        l = scal[3:4, :] * jnp.exp(m2 - m_f) + jnp.exp(e_self - m_f)
        acc = stats[1:2, :] * jnp.exp(m2 - m_f) + jnp.exp(e_self - m_f) * h2_0
        agg2_0 = acc / (l + EPS)
        t0 = _tm(agg2_0)

        ssum = stats[2:3, :] + agg2_0
        tsum = scal[6:7, :] + t0
        m_s = ssum / (SEQ + 1)
        m_t = tsum / (SEQ + 1)
        inner = -(m_t * m_t) + jnp.sum(m_s * m_s, axis=1, keepdims=True)
        denom = jnp.sqrt(jnp.clip(-inner, 1e-8, None))
        gm_ref[0] = jnp.concatenate([m_t, m_s], axis=1) / denom

        y = t0 * linT_ref[...] + jnp.dot(agg2_0, linS_ref[...],
                                         preferred_element_type=jnp.float32)
        osp = y + linb_ref[...] + ps
        out_ref[0] = jnp.concatenate([_tm(osp), osp], axis=1)


def kernel(hidden_states, pooled_output, proj_W1, proj_b1, proj_W2, proj_b2,
           gat1_W, gat1_a, gat2_W, gat2_a, lin_W, lin_b):
    f32 = jnp.float32
    ps = pooled_output[:, 1:].reshape(BS, 1, 512)
    b1 = proj_b1.reshape(1, -1)
    W2t = proj_W2[0:1, :]
    W2s = proj_W2[1:, :]
    b2 = proj_b2.reshape(1, -1)
    g1Wt = gat1_W[0:1, :]
    g1Ws = gat1_W[1:, :]
    a1d = gat1_a[:384].reshape(1, -1)
    a1s = gat1_a[384:].reshape(1, -1)
    g2Wt = gat2_W[0:1, :]
    g2Ws = gat2_W[1:, :]
    a2d = gat2_a[:384].reshape(1, -1)
    a2s = gat2_a[384:].reshape(1, -1)
    linT = lin_W[0:1, :]
    linS = lin_W[1:, :]
    linb = lin_b.reshape(1, -1)

    full = lambda arr: pl.BlockSpec(arr.shape, lambda b, s: (0,) * arr.ndim)
    in_specs = [
        pl.BlockSpec((1, 1, TILE, 769),
                     lambda b, s: (0, b, jnp.minimum(s, NT - 1), 0)),
        pl.BlockSpec((1, 1, 512), lambda b, s: (b, 0, 0)),
        full(proj_W1), full(b1), full(W2t), full(W2s), full(b2),
        full(g1Wt), full(g1Ws), full(a1d), full(a1s),
        full(g2Wt), full(g2Ws), full(a2d), full(a2s),
        full(linT), full(linS), full(linb),
    ]
    out_specs = (
        pl.BlockSpec((1, 1, 513), lambda b, s: (b, 0, 0)),
        pl.BlockSpec((1, 1, 385), lambda b, s: (b, 0, 0)),
    )
    out, gm = pl.pallas_call(
        _body,
        grid=(BS, STEPS),
        in_specs=in_specs,
        out_specs=out_specs,
        out_shape=(
            jax.ShapeDtypeStruct((BS, 1, 513), f32),
            jax.ShapeDtypeStruct((BS, 1, 385), f32),
        ),
        scratch_shapes=[
            pltpu.VMEM((SEQ, 384), f32),
            pltpu.VMEM((SEQ, 1), f32),
            pltpu.VMEM((SEQ, 1), f32),
            pltpu.VMEM((8, 384), f32),
            pltpu.VMEM((8, 1), f32),
        ],
    )(hidden_states, ps, proj_W1, b1, W2t, W2s, b2,
      g1Wt, g1Ws, a1d, a1s, g2Wt, g2Ws, a2d, a2s, linT, linS, linb)
    return (out.reshape(BS, 513), gm.reshape(BS, 385))


# speedup vs baseline: 22.9871x; 1.0325x over previous
import jax
import jax.numpy as jnp
from jax.experimental import pallas as pl
from jax.experimental.pallas import tpu as pltpu

BS = 4
SEQ = 4096
TILE = 2048
NT = SEQ // TILE
STEPS = 2 * NT + 1
EPS = 1e-16
NEG = -1e30


def _tm(x):
    return jnp.sqrt(1.0 + jnp.sum(x * x, axis=-1, keepdims=True))


def _lrelu(x):
    return jnp.where(x >= 0, x, 0.2 * x)


def _body(hs_ref, ps_ref, W1_ref, b1_ref, W2t_ref, W2s_ref, b2_ref,
          g1Wt_ref, g1Ws_ref, a1d_ref, a1s_ref,
          g2Wt_ref, g2Ws_ref, a2d_ref, a2s_ref,
          linT_ref, linS_ref, linb_ref,
          out_ref, gm_ref,
          h2_buf, u2_buf, v2_buf, stats, scal):
    s = pl.program_id(1)

    ps = ps_ref[0]
    pt = _tm(ps)

    g1Wt = g1Wt_ref[...]
    g2Wt = g2Wt_ref[...]
    a1d = a1d_ref[...]
    a1s = a1s_ref[...]
    a2d = a2d_ref[...]
    a2s = a2s_ref[...]

    h1_0 = pt * g1Wt + jnp.dot(ps, g1Ws_ref[...])
    u1_0 = jnp.sum(h1_0 * a1d, axis=1, keepdims=True)
    v1_0 = jnp.sum(h1_0 * a1s, axis=1, keepdims=True)

    @pl.when(s == 0)
    def _init1():
        scal[0:1, :] = jnp.full((1, 1), NEG, jnp.float32)
        scal[1:2, :] = jnp.zeros((1, 1), jnp.float32)
        stats[0:1, :] = jnp.zeros((1, 384), jnp.float32)

    @pl.when(s < NT)
    def _pass1():
        x = hs_ref[0, 0]
        y1 = jnp.dot(x, W1_ref[...], preferred_element_type=jnp.float32)
        y1 = y1 + b1_ref[...]
        g = jax.nn.gelu(y1)
        tg = _tm(g)
        y2 = tg * W2t_ref[...] + jnp.dot(g, W2s_ref[...],
                                         preferred_element_type=jnp.float32)
        y2 = y2 + b2_ref[...]
        t2 = _tm(y2)
        h1 = t2 * g1Wt + jnp.dot(y2, g1Ws_ref[...],
                                 preferred_element_type=jnp.float32)
        u1 = jnp.sum(h1 * a1d, axis=1, keepdims=True)
        v1 = jnp.sum(h1 * a1s, axis=1, keepdims=True)

        e0 = _lrelu(u1 + v1_0)
        es = _lrelu(u1 + v1)
        mm = jnp.maximum(e0, es)
        w0 = jnp.exp(e0 - mm)
        ws = jnp.exp(es - mm)
        agg1 = (w0 * h1_0 + ws * h1) / (w0 + ws + EPS)

        z = jax.nn.gelu(agg1)
        tz = _tm(z)
        h2 = tz * g2Wt + jnp.dot(z, g2Ws_ref[...],
                                 preferred_element_type=jnp.float32)
        u2 = jnp.sum(h2 * a2d, axis=1, keepdims=True)
        v2 = jnp.sum(h2 * a2s, axis=1, keepdims=True)

        off = s * TILE
        h2_buf[pl.ds(off, TILE), :] = h2
        u2_buf[pl.ds(off, TILE), :] = u2
        v2_buf[pl.ds(off, TILE), :] = v2

        sc = _lrelu(u1_0 + v1)
        mt = jnp.max(sc, keepdims=True)
        m_old = scal[0:1, :]
        m_new = jnp.maximum(m_old, mt)
        c = jnp.exp(m_old - m_new)
        p = jnp.exp(sc - m_new)
        scal[0:1, :] = m_new
        scal[1:2, :] = scal[1:2, :] * c + jnp.sum(p, keepdims=True)
        stats[0:1, :] = stats[0:1, :] * c + jax.lax.dot_general(
            p, h1, (((0,), (0,)), ((), ())),
            preferred_element_type=jnp.float32)

    @pl.when(s == NT)
    def _hub():
        e_self = _lrelu(u1_0 + v1_0)
        m1 = scal[0:1, :]
        m_f = jnp.maximum(m1, e_self)
        l = scal[1:2, :] * jnp.exp(m1 - m_f) + jnp.exp(e_self - m_f)
        acc = stats[0:1, :] * jnp.exp(m1 - m_f) + jnp.exp(e_self - m_f) * h1_0
        agg1_0 = acc / (l + EPS)

        z0 = jax.nn.gelu(agg1_0)
        tz0 = _tm(z0)
        h2_0 = tz0 * g2Wt + jnp.dot(z0, g2Ws_ref[...],
                                    preferred_element_type=jnp.float32)
        stats[3:4, :] = h2_0
        scal[4:5, :] = jnp.sum(h2_0 * a2d, axis=1, keepdims=True)
        scal[5:6, :] = jnp.sum(h2_0 * a2s, axis=1, keepdims=True)

        scal[2:3, :] = jnp.full((1, 1), NEG, jnp.float32)
        scal[3:4, :] = jnp.zeros((1, 1), jnp.float32)
        scal[6:7, :] = jnp.zeros((1, 1), jnp.float32)
        stats[1:2, :] = jnp.zeros((1, 384), jnp.float32)
        stats[2:3, :] = jnp.zeros((1, 384), jnp.float32)

    @pl.when(s > NT)
    def _pass2():
        off = (s - NT - 1) * TILE
        h2 = h2_buf[pl.ds(off, TILE), :]
        u2 = u2_buf[pl.ds(off, TILE), :]
        v2 = v2_buf[pl.ds(off, TILE), :]
        h2_0 = stats[3:4, :]
        u2_0 = scal[4:5, :]
        v2_0 = scal[5:6, :]

        sc = _lrelu(u2_0 + v2)
        mt = jnp.max(sc, keepdims=True)
        m_old = scal[2:3, :]
        m_new = jnp.maximum(m_old, mt)
        c = jnp.exp(m_old - m_new)
        p = jnp.exp(sc - m_new)
        scal[2:3, :] = m_new
        scal[3:4, :] = scal[3:4, :] * c + jnp.sum(p, keepdims=True)
        stats[1:2, :] = stats[1:2, :] * c + jax.lax.dot_general(
            p, h2, (((0,), (0,)), ((), ())),
            preferred_element_type=jnp.float32)

        e0 = _lrelu(u2 + v2_0)
        es = _lrelu(u2 + v2)
        mm = jnp.maximum(e0, es)
        w0 = jnp.exp(e0 - mm)
        ws = jnp.exp(es - mm)
        agg2 = (w0 * h2_0 + ws * h2) / (w0 + ws + EPS)
        tt = _tm(agg2)
        stats[2:3, :] = stats[2:3, :] + jnp.sum(agg2, axis=0, keepdims=True)
        scal[6:7, :] = scal[6:7, :] + jnp.sum(tt, keepdims=True)

    @pl.when(s == STEPS - 1)
    def _final():
        h2_0 = stats[3:4, :]
        u2_0 = scal[4:5, :]
        v2_0 = scal[5:6, :]
        e_self = _lrelu(u2_0 + v2_0)
        m2 = scal[2:3, :]
        m_f = jnp.maximum(m2, e_self)
        l = scal[3:4, :] * jnp.exp(m2 - m_f) + jnp.exp(e_self - m_f)
        acc = stats[1:2, :] * jnp.exp(m2 - m_f) + jnp.exp(e_self - m_f) * h2_0
        agg2_0 = acc / (l + EPS)
        t0 = _tm(agg2_0)

        ssum = stats[2:3, :] + agg2_0
        tsum = scal[6:7, :] + t0
        m_s = ssum / (SEQ + 1)
        m_t = tsum / (SEQ + 1)
        inner = -(m_t * m_t) + jnp.sum(m_s * m_s, axis=1, keepdims=True)
        denom = jnp.sqrt(jnp.clip(-inner, 1e-8, None))
        gm_ref[0] = jnp.concatenate([m_t, m_s], axis=1) / denom

        y = t0 * linT_ref[...] + jnp.dot(agg2_0, linS_ref[...],
                                         preferred_element_type=jnp.float32)
        osp = y + linb_ref[...] + ps
        out_ref[0] = jnp.concatenate([_tm(osp), osp], axis=1)


def kernel(hidden_states, pooled_output, proj_W1, proj_b1, proj_W2, proj_b2,
           gat1_W, gat1_a, gat2_W, gat2_a, lin_W, lin_b):
    f32 = jnp.float32
    ps = pooled_output[:, 1:].reshape(BS, 1, 512)
    b1 = proj_b1.reshape(1, -1)
    W2t = proj_W2[0:1, :]
    W2s = proj_W2[1:, :]
    b2 = proj_b2.reshape(1, -1)
    g1Wt = gat1_W[0:1, :]
    g1Ws = gat1_W[1:, :]
    a1d = gat1_a[:384].reshape(1, -1)
    a1s = gat1_a[384:].reshape(1, -1)
    g2Wt = gat2_W[0:1, :]
    g2Ws = gat2_W[1:, :]
    a2d = gat2_a[:384].reshape(1, -1)
    a2s = gat2_a[384:].reshape(1, -1)
    linT = lin_W[0:1, :]
    linS = lin_W[1:, :]
    linb = lin_b.reshape(1, -1)

    full = lambda arr: pl.BlockSpec(arr.shape, lambda b, s: (0,) * arr.ndim)
    in_specs = [
        pl.BlockSpec((1, 1, TILE, 769),
                     lambda b, s: (0, b, jnp.minimum(s, NT - 1), 0)),
        pl.BlockSpec((1, 1, 512), lambda b, s: (b, 0, 0)),
        full(proj_W1), full(b1), full(W2t), full(W2s), full(b2),
        full(g1Wt), full(g1Ws), full(a1d), full(a1s),
        full(g2Wt), full(g2Ws), full(a2d), full(a2s),
        full(linT), full(linS), full(linb),
    ]
    out_specs = (
        pl.BlockSpec((1, 1, 513), lambda b, s: (b, 0, 0)),
        pl.BlockSpec((1, 1, 385), lambda b, s: (b, 0, 0)),
    )
    out, gm = pl.pallas_call(
        _body,
        grid=(BS, STEPS),
        in_specs=in_specs,
        out_specs=out_specs,
        out_shape=(
            jax.ShapeDtypeStruct((BS, 1, 513), f32),
            jax.ShapeDtypeStruct((BS, 1, 385), f32),
        ),
        scratch_shapes=[
            pltpu.VMEM((SEQ, 384), f32),
            pltpu.VMEM((SEQ, 1), f32),
            pltpu.VMEM((SEQ, 1), f32),
            pltpu.VMEM((8, 384), f32),
            pltpu.VMEM((8, 1), f32),
        ],
    )(hidden_states, ps, proj_W1, b1, W2t, W2s, b2,
      g1Wt, g1Ws, a1d, a1s, g2Wt, g2Ws, a2d, a2s, linT, linS, linb)
    return (out.reshape(BS, 513), gm.reshape(BS, 385))
